# baseline (speedup 1.0000x reference)
import jax
import jax.numpy as jnp
from jax.experimental import pallas as pl


def _body(x_ref, s_ref, o_ref):
    o_ref[...] = jnp.dot(x_ref[...], s_ref[...], preferred_element_type=jnp.float32)


def kernel(joints, joint_maps):
    B, J, C = joints.shape
    K = joint_maps.shape[0]
    x = joints.reshape(B, J * C)
    cols = (joint_maps.astype(jnp.int32)[:, None] * C
            + jnp.arange(C, dtype=jnp.int32)[None, :]).reshape(K * C)
    sel = (jnp.arange(J * C, dtype=jnp.int32)[:, None] == cols[None, :]).astype(jnp.float32)

    BB = 2048
    out = pl.pallas_call(
        _body,
        grid=(B // BB,),
        in_specs=[
            pl.BlockSpec((BB, J * C), lambda i: (i, 0)),
            pl.BlockSpec((J * C, K * C), lambda i: (0, 0)),
        ],
        out_specs=pl.BlockSpec((BB, K * C), lambda i: (i, 0)),
        out_shape=jax.ShapeDtypeStruct((B, K * C), jnp.float32),
    )(x, sel)
    return out.reshape(B, K, C)
